# initial kernel scaffold (unmeasured)
import jax
import jax.numpy as jnp
from jax import lax
from jax.experimental import pallas as pl
from jax.experimental.pallas import tpu as pltpu


def kernel(
    x,
):
    def body(*refs):
        pass

    out_shape = jax.ShapeDtypeStruct(..., jnp.float32)
    return pl.pallas_call(body, out_shape=out_shape)(...)


# baseline (device time: 8174 ns/iter reference)
import jax
import jax.numpy as jnp
from jax import lax
from jax.experimental import pallas as pl
from jax.experimental.pallas import tpu as pltpu


def kernel(x):
    m, n = x.shape

    def body(x_ref, out_ref, row_halo, col_halo, row_stage, col_stage,
             row_send_sem, row_recv_sem, col_send_sem, col_recv_sem):
        mx = lax.axis_index("x")
        my = lax.axis_index("y")

        barrier_sem = pltpu.get_barrier_semaphore()
        pl.semaphore_signal(barrier_sem, inc=1, device_id=(1 - mx, my),
                            device_id_type=pl.DeviceIdType.MESH)
        pl.semaphore_signal(barrier_sem, inc=1, device_id=(mx, 1 - my),
                            device_id_type=pl.DeviceIdType.MESH)
        pl.semaphore_wait(barrier_sem, 2)

        @pl.when(mx == 0)
        def _():
            row_stage[:, :] = x_ref[m - 1:m, :]

        @pl.when(mx == 1)
        def _():
            row_stage[:, :] = x_ref[0:1, :]

        @pl.when(my == 0)
        def _():
            col_stage[:, :] = x_ref[:, n - 1:n]

        @pl.when(my == 1)
        def _():
            col_stage[:, :] = x_ref[:, 0:1]

        row_rdma = pltpu.make_async_remote_copy(
            src_ref=row_stage,
            dst_ref=row_halo,
            send_sem=row_send_sem,
            recv_sem=row_recv_sem,
            device_id=(1 - mx, my),
            device_id_type=pl.DeviceIdType.MESH,
        )
        col_rdma = pltpu.make_async_remote_copy(
            src_ref=col_stage,
            dst_ref=col_halo,
            send_sem=col_send_sem,
            recv_sem=col_recv_sem,
            device_id=(mx, 1 - my),
            device_id_type=pl.DeviceIdType.MESH,
        )
        row_rdma.start()
        col_rdma.start()
        row_rdma.wait()
        col_rdma.wait()

        xv = x_ref[:, :]
        rh = row_halo[:, :]
        ch = col_halo[:, :]

        n_arr = jnp.concatenate([rh, xv[:-1, :]], axis=0)
        s_arr = jnp.concatenate([xv[1:, :], rh], axis=0)
        w_arr = jnp.concatenate([ch, xv[:, :-1]], axis=1)
        e_arr = jnp.concatenate([xv[:, 1:], ch], axis=1)
        stencil = 0.5 * xv + 0.125 * (n_arr + s_arr + w_arr + e_arr)

        rows = lax.broadcasted_iota(jnp.int32, (m, n), 0)
        cols = lax.broadcasted_iota(jnp.int32, (m, n), 1)
        boundary = (
            ((mx == 0) & (rows == 0))
            | ((mx == 1) & (rows == m - 1))
            | ((my == 0) & (cols == 0))
            | ((my == 1) & (cols == n - 1))
        )
        out_ref[:, :] = jnp.where(boundary, xv, stencil)

    return pl.pallas_call(
        body,
        out_shape=jax.ShapeDtypeStruct((m, n), x.dtype),
        in_specs=[pl.BlockSpec(memory_space=pltpu.VMEM)],
        out_specs=pl.BlockSpec(memory_space=pltpu.VMEM),
        scratch_shapes=[
            pltpu.VMEM((1, n), x.dtype),
            pltpu.VMEM((m, 1), x.dtype),
            pltpu.VMEM((1, n), x.dtype),
            pltpu.VMEM((m, 1), x.dtype),
            pltpu.SemaphoreType.DMA,
            pltpu.SemaphoreType.DMA,
            pltpu.SemaphoreType.DMA,
            pltpu.SemaphoreType.DMA,
        ],
        compiler_params=pltpu.CompilerParams(collective_id=0),
    )(x)


# device time: 8162 ns/iter; 1.0015x vs baseline; 1.0015x over previous
import jax
import jax.numpy as jnp
from jax import lax
from jax.experimental import pallas as pl
from jax.experimental.pallas import tpu as pltpu


def kernel(x):
    m, n = x.shape

    def body(x_ref, out_ref, row_halo, col_halo, row_stage, col_stage,
             row_send_sem, row_recv_sem, col_send_sem, col_recv_sem):
        mx = lax.axis_index("x")
        my = lax.axis_index("y")

        barrier_sem = pltpu.get_barrier_semaphore()
        pl.semaphore_signal(barrier_sem, inc=1, device_id=(1 - mx, my),
                            device_id_type=pl.DeviceIdType.MESH)
        pl.semaphore_signal(barrier_sem, inc=1, device_id=(mx, 1 - my),
                            device_id_type=pl.DeviceIdType.MESH)
        pl.semaphore_wait(barrier_sem, 2)

        @pl.when(mx == 0)
        def _():
            row_stage[:, :] = x_ref[m - 1:m, :]

        @pl.when(mx == 1)
        def _():
            row_stage[:, :] = x_ref[0:1, :]

        @pl.when(my == 0)
        def _():
            col_stage[:, :] = x_ref[:, n - 1:n]

        @pl.when(my == 1)
        def _():
            col_stage[:, :] = x_ref[:, 0:1]

        row_rdma = pltpu.make_async_remote_copy(
            src_ref=row_stage,
            dst_ref=row_halo,
            send_sem=row_send_sem,
            recv_sem=row_recv_sem,
            device_id=(1 - mx, my),
            device_id_type=pl.DeviceIdType.MESH,
        )
        col_rdma = pltpu.make_async_remote_copy(
            src_ref=col_stage,
            dst_ref=col_halo,
            send_sem=col_send_sem,
            recv_sem=col_recv_sem,
            device_id=(mx, 1 - my),
            device_id_type=pl.DeviceIdType.MESH,
        )
        row_rdma.start()
        col_rdma.start()

        xv = x_ref[:, :]
        n_arr = jnp.concatenate([xv[0:1, :], xv[:-1, :]], axis=0)
        s_arr = jnp.concatenate([xv[1:, :], xv[m - 1:m, :]], axis=0)
        w_arr = jnp.concatenate([xv[:, 0:1], xv[:, :-1]], axis=1)
        e_arr = jnp.concatenate([xv[:, 1:], xv[:, n - 1:n]], axis=1)
        out_ref[:, :] = 0.5 * xv + 0.125 * (n_arr + s_arr + w_arr + e_arr)

        row_rdma.wait()
        col_rdma.wait()

        def patch_row(r, north, south):
            row_x = x_ref[r:r + 1, :]
            ch = col_halo[r:r + 1, :]
            w_row = jnp.concatenate([ch, row_x[:, :n - 1]], axis=1)
            e_row = jnp.concatenate([row_x[:, 1:], ch], axis=1)
            out_ref[r:r + 1, :] = (
                0.5 * row_x + 0.125 * (north + south + w_row + e_row))

        @pl.when(mx == 0)
        def _():
            patch_row(m - 1, x_ref[m - 2:m - 1, :], row_halo[:, :])

        @pl.when(mx == 1)
        def _():
            patch_row(0, row_halo[:, :], x_ref[1:2, :])

        def patch_col(c, west, east):
            col_x = x_ref[:, c:c + 1]
            rh = row_halo[0:1, c:c + 1]
            n_col = jnp.concatenate([rh, col_x[:m - 1, :]], axis=0)
            s_col = jnp.concatenate([col_x[1:, :], rh], axis=0)
            out_ref[:, c:c + 1] = (
                0.5 * col_x + 0.125 * (n_col + s_col + west + east))

        @pl.when(my == 0)
        def _():
            patch_col(n - 1, x_ref[:, n - 2:n - 1], col_halo[:, :])

        @pl.when(my == 1)
        def _():
            patch_col(0, col_halo[:, :], x_ref[:, 1:2])

        @pl.when(mx == 0)
        def _():
            out_ref[0:1, :] = x_ref[0:1, :]

        @pl.when(mx == 1)
        def _():
            out_ref[m - 1:m, :] = x_ref[m - 1:m, :]

        @pl.when(my == 0)
        def _():
            out_ref[:, 0:1] = x_ref[:, 0:1]

        @pl.when(my == 1)
        def _():
            out_ref[:, n - 1:n] = x_ref[:, n - 1:n]

    return pl.pallas_call(
        body,
        out_shape=jax.ShapeDtypeStruct((m, n), x.dtype),
        in_specs=[pl.BlockSpec(memory_space=pltpu.VMEM)],
        out_specs=pl.BlockSpec(memory_space=pltpu.VMEM),
        scratch_shapes=[
            pltpu.VMEM((1, n), x.dtype),
            pltpu.VMEM((m, 1), x.dtype),
            pltpu.VMEM((1, n), x.dtype),
            pltpu.VMEM((m, 1), x.dtype),
            pltpu.SemaphoreType.DMA,
            pltpu.SemaphoreType.DMA,
            pltpu.SemaphoreType.DMA,
            pltpu.SemaphoreType.DMA,
        ],
        compiler_params=pltpu.CompilerParams(collective_id=0),
    )(x)


# device time: 6838 ns/iter; 1.1954x vs baseline; 1.1936x over previous
import jax
import jax.numpy as jnp
from jax import lax
from jax.experimental import pallas as pl
from jax.experimental.pallas import tpu as pltpu


def kernel(x):
    m, n = x.shape

    def body(x_ref, out_ref, row_halo, col_halo, row_stage, col_stage,
             row_send_sem, row_recv_sem, col_send_sem, col_recv_sem):
        mx = lax.axis_index("x")
        my = lax.axis_index("y")

        barrier_sem = pltpu.get_barrier_semaphore()
        pl.semaphore_signal(barrier_sem, inc=1, device_id=(1 - mx, my),
                            device_id_type=pl.DeviceIdType.MESH)
        pl.semaphore_signal(barrier_sem, inc=1, device_id=(mx, 1 - my),
                            device_id_type=pl.DeviceIdType.MESH)

        @pl.when(mx == 0)
        def _():
            row_stage[:, :] = x_ref[m - 1:m, :]

        @pl.when(mx == 1)
        def _():
            row_stage[:, :] = x_ref[0:1, :]

        @pl.when(my == 0)
        def _():
            col_stage[:, :] = jnp.transpose(x_ref[:, n - 1:n], (1, 0))

        @pl.when(my == 1)
        def _():
            col_stage[:, :] = jnp.transpose(x_ref[:, 0:1], (1, 0))

        pl.semaphore_wait(barrier_sem, 2)

        row_rdma = pltpu.make_async_remote_copy(
            src_ref=row_stage,
            dst_ref=row_halo,
            send_sem=row_send_sem,
            recv_sem=row_recv_sem,
            device_id=(1 - mx, my),
            device_id_type=pl.DeviceIdType.MESH,
        )
        col_rdma = pltpu.make_async_remote_copy(
            src_ref=col_stage,
            dst_ref=col_halo,
            send_sem=col_send_sem,
            recv_sem=col_recv_sem,
            device_id=(mx, 1 - my),
            device_id_type=pl.DeviceIdType.MESH,
        )
        row_rdma.start()
        col_rdma.start()

        xv = x_ref[:, :]
        n_arr = jnp.concatenate([xv[0:1, :], xv[:-1, :]], axis=0)
        s_arr = jnp.concatenate([xv[1:, :], xv[m - 1:m, :]], axis=0)
        w_arr = jnp.concatenate([xv[:, 0:1], xv[:, :-1]], axis=1)
        e_arr = jnp.concatenate([xv[:, 1:], xv[:, n - 1:n]], axis=1)
        out_ref[:, :] = 0.5 * xv + 0.125 * (n_arr + s_arr + w_arr + e_arr)

        row_rdma.wait()
        col_rdma.wait()

        ch_full = jnp.transpose(col_halo[:, :], (1, 0))

        def patch_row(r, north, south):
            row_x = x_ref[r:r + 1, :]
            ch = col_halo[:, r:r + 1]
            w_row = jnp.concatenate([ch, row_x[:, :n - 1]], axis=1)
            e_row = jnp.concatenate([row_x[:, 1:], ch], axis=1)
            out_ref[r:r + 1, :] = (
                0.5 * row_x + 0.125 * (north + south + w_row + e_row))

        @pl.when(mx == 0)
        def _():
            patch_row(m - 1, x_ref[m - 2:m - 1, :], row_halo[:, :])

        @pl.when(mx == 1)
        def _():
            patch_row(0, row_halo[:, :], x_ref[1:2, :])

        def patch_col(c, west, east):
            col_x = x_ref[:, c:c + 1]
            rh = row_halo[0:1, c:c + 1]
            n_col = jnp.concatenate([rh, col_x[:m - 1, :]], axis=0)
            s_col = jnp.concatenate([col_x[1:, :], rh], axis=0)
            out_ref[:, c:c + 1] = (
                0.5 * col_x + 0.125 * (n_col + s_col + west + east))

        @pl.when(my == 0)
        def _():
            patch_col(n - 1, x_ref[:, n - 2:n - 1], ch_full)

        @pl.when(my == 1)
        def _():
            patch_col(0, ch_full, x_ref[:, 1:2])

        @pl.when(mx == 0)
        def _():
            out_ref[0:1, :] = x_ref[0:1, :]

        @pl.when(mx == 1)
        def _():
            out_ref[m - 1:m, :] = x_ref[m - 1:m, :]

        @pl.when(my == 0)
        def _():
            out_ref[:, 0:1] = x_ref[:, 0:1]

        @pl.when(my == 1)
        def _():
            out_ref[:, n - 1:n] = x_ref[:, n - 1:n]

    return pl.pallas_call(
        body,
        out_shape=jax.ShapeDtypeStruct((m, n), x.dtype),
        in_specs=[pl.BlockSpec(memory_space=pltpu.VMEM)],
        out_specs=pl.BlockSpec(memory_space=pltpu.VMEM),
        scratch_shapes=[
            pltpu.VMEM((1, n), x.dtype),
            pltpu.VMEM((1, m), x.dtype),
            pltpu.VMEM((1, n), x.dtype),
            pltpu.VMEM((1, m), x.dtype),
            pltpu.SemaphoreType.DMA,
            pltpu.SemaphoreType.DMA,
            pltpu.SemaphoreType.DMA,
            pltpu.SemaphoreType.DMA,
        ],
        compiler_params=pltpu.CompilerParams(collective_id=0),
    )(x)
